# baseline (speedup 1.0000x reference)
import functools

import jax
import jax.numpy as jnp
from jax import lax
from jax.experimental import pallas as pl
from jax.experimental.pallas import tpu as pltpu
from jax.experimental.pallas import tpu_sc as plsc

N = 10000
E = 160000
D = 256
B = 64
EPS = 1e-5

NC = 2
NS = 16
LANES = 16
HALF = D // NC

EPT = E // NS
CHUNK = 80
NCHUNK = EPT // CHUNK
NPAD = 10240
ROWS = NPAD // NS
WCHUNK = 128
NWC = ROWS // WCHUNK

NB = 400
NBLK = N // NB


def _sc_body(src_hbm, dst_hbm, x2_hbm, ea2_hbm, out_hbm,
             src_v, dst_v, xi_v, ei_v, xrows_v, ea_v, zrow_v, acc_sh,
             sem_x, sem_e):
    c = lax.axis_index("c")
    s = lax.axis_index("s")

    zero16 = jnp.zeros((LANES,), jnp.float32)

    def zrow(r, carry):
        for j in range(HALF // LANES):
            zrow_v[r, pl.ds(j * LANES, LANES)] = zero16
        return carry

    lax.fori_loop(0, WCHUNK, zrow, 0)
    row0 = s * ROWS
    for k in range(NWC):
        pltpu.sync_copy(zrow_v, acc_sh.at[pl.ds(row0 + k * WCHUNK, WCHUNK)])
    plsc.subcore_barrier()

    lane2 = lax.iota(jnp.int32, LANES) * 2
    e0 = s * EPT

    def chunk_body(k, carry):
        base = e0 + k * CHUNK
        pltpu.sync_copy(src_hbm.at[pl.ds(base, CHUNK)], src_v)
        pltpu.sync_copy(dst_hbm.at[pl.ds(base, CHUNK)], dst_v)
        for q in range(CHUNK // LANES):
            sl = pl.ds(q * LANES, LANES)
            xi_v[sl] = src_v[sl] * 2 + c
            ei_v[sl] = lane2 + (2 * (base + q * LANES) + c)
        cp_x = pltpu.async_copy(x2_hbm.at[xi_v], xrows_v, sem_x)
        cp_e = pltpu.async_copy(ea2_hbm.at[ei_v], ea_v, sem_e)
        cp_x.wait()
        cp_e.wait()

        def rowf(r, rc):
            for j in range(HALF // LANES):
                sl = pl.ds(j * LANES, LANES)
                xrows_v[r, sl] = jnp.maximum(xrows_v[r, sl] + ea_v[r, sl], 0.0)
            return rc

        lax.fori_loop(0, CHUNK, rowf, 0)
        pltpu.sync_copy(xrows_v, acc_sh.at[dst_v], add=True)
        return carry

    lax.fori_loop(0, NCHUNK, chunk_body, 0)
    plsc.subcore_barrier()

    for k in range(NWC):
        sl = pl.ds(row0 + k * WCHUNK, WCHUNK)
        pltpu.sync_copy(acc_sh.at[sl], out_hbm.at[c, sl])


@functools.lru_cache(maxsize=None)
def _build_sc_aggregate():
    return pl.kernel(
        _sc_body,
        out_type=jax.ShapeDtypeStruct((NC, NPAD, HALF), jnp.float32),
        mesh=plsc.VectorSubcoreMesh(
            core_axis_name="c", subcore_axis_name="s",
            num_cores=NC, num_subcores=NS),
        scratch_types=[
            pltpu.VMEM((CHUNK,), jnp.int32),
            pltpu.VMEM((CHUNK,), jnp.int32),
            pltpu.VMEM((CHUNK,), jnp.int32),
            pltpu.VMEM((CHUNK,), jnp.int32),
            pltpu.VMEM((CHUNK, HALF), jnp.float32),
            pltpu.VMEM((CHUNK, HALF), jnp.float32),
            pltpu.VMEM((WCHUNK, HALF), jnp.float32),
            pltpu.VMEM_SHARED((NPAD, HALF), jnp.float32),
            pltpu.SemaphoreType.DMA,
            pltpu.SemaphoreType.DMA,
        ],
    )


def _sc_aggregate(src, dst, x2, ea2):
    return _build_sc_aggregate()(src, dst, x2, ea2)


def _mlp_stats_kernel(x_ref, agg_ref, batch_ref, w1_ref, b1_ref, g_ref,
                      be_ref, mu_ref, va_ref, w2_ref, b2_ref,
                      out_ref, stats_ref):
    i = pl.program_id(0)
    x = x_ref[...]
    h = x + jnp.concatenate([agg_ref[0], agg_ref[1]], axis=1)
    h1 = jnp.dot(h, w1_ref[...], preferred_element_type=jnp.float32)
    scale = g_ref[...] * lax.rsqrt(va_ref[...] + EPS)
    h1 = (h1 + b1_ref[...] - mu_ref[...]) * scale + be_ref[...]
    h1 = jnp.maximum(h1, 0.0)
    out = jnp.dot(h1, w2_ref[...], preferred_element_type=jnp.float32)
    out = out + b2_ref[...]
    out_ref[...] = out

    batch_col = batch_ref[0, 0, :].reshape(NB, 1)
    iota_b = lax.broadcasted_iota(jnp.int32, (NB, B), 1)
    p = (batch_col == iota_b).astype(jnp.float32)
    m1 = lax.dot_general(p, out, (((0,), (0,)), ((), ())),
                         preferred_element_type=jnp.float32)
    m2 = lax.dot_general(p, out * out, (((0,), (0,)), ((), ())),
                         preferred_element_type=jnp.float32)
    deg = jnp.broadcast_to(jnp.sum(p, axis=0)[:, None], (B, D))
    stacked = jnp.stack([m1, m2, deg])

    @pl.when(i == 0)
    def _():
        stats_ref[...] = stacked

    @pl.when(i > 0)
    def _():
        stats_ref[...] = stats_ref[...] + stacked


def _final_kernel(x_ref, out_in_ref, batch_ref, stats_ref, lnw_ref, lnb_ref,
                  gnw_ref, gnb_ref, gns_ref, res_ref):
    m1 = stats_ref[0]
    m2 = stats_ref[1]
    deg = stats_ref[2, :, 0:1]
    cnt = jnp.maximum(deg, 1.0)
    norm = cnt * D
    ms1 = jnp.sum(m1, axis=1, keepdims=True)
    ms2 = jnp.sum(m2, axis=1, keepdims=True)
    m = ms1 / norm
    varb = ms2 / norm - m * m
    inv_s = lax.rsqrt(varb + EPS)
    lnw = lnw_ref[...][None, :]
    gns = gns_ref[...][None, :]
    gnw = gnw_ref[...][None, :]
    a = lnw * inv_s
    cc = lnb_ref[...][None, :] - m * a
    mu1 = m1 / cnt
    mu2 = m2 / cnt
    beta = cc * (1.0 - gns) - a * mu1 * gns
    gvar = a * a * mu2 + 2.0 * a * beta * mu1 + beta * beta
    invt = lax.rsqrt(gvar + EPS)
    gamma = gnw * a * invt
    delta = gnw * beta * invt + gnb_ref[...][None, :]

    batch_col = batch_ref[0, 0, :].reshape(NB, 1)
    iota_b = lax.broadcasted_iota(jnp.int32, (NB, B), 1)
    p = (batch_col == iota_b).astype(jnp.float32)
    gn = jnp.dot(p, gamma, preferred_element_type=jnp.float32)
    dn = jnp.dot(p, delta, preferred_element_type=jnp.float32)
    res_ref[...] = x_ref[...] + jnp.maximum(gn * out_in_ref[...] + dn, 0.0)


def _full(shape):
    nd = len(shape)
    return pl.BlockSpec(shape, lambda i: (0,) * nd)


def kernel(x, edge_index, edge_attr, batch, W1, b1, bn_gamma, bn_beta,
           bn_mean, bn_var, W2, b2, ln_w, ln_b, gn_w, gn_b, gn_scale):
    src = edge_index[0]
    dst = edge_index[1]
    x2 = x.reshape(2 * N, HALF)
    ea2 = edge_attr.reshape(2 * E, HALF)
    agg = _sc_aggregate(src, dst, x2, ea2)

    batch3 = batch.reshape(NBLK, 1, NB)
    blk = pl.BlockSpec((NB, D), lambda i: (i, 0))
    bblk = pl.BlockSpec((1, 1, NB), lambda i: (i, 0, 0))

    out, stats = pl.pallas_call(
        _mlp_stats_kernel,
        grid=(NBLK,),
        in_specs=[
            blk,
            pl.BlockSpec((NC, NB, HALF), lambda i: (0, i, 0)),
            bblk,
            _full((D, 2 * D)), _full((2 * D,)), _full((2 * D,)),
            _full((2 * D,)), _full((2 * D,)), _full((2 * D,)),
            _full((2 * D, D)), _full((D,)),
        ],
        out_specs=[
            blk,
            pl.BlockSpec((3, B, D), lambda i: (0, 0, 0)),
        ],
        out_shape=[
            jax.ShapeDtypeStruct((N, D), jnp.float32),
            jax.ShapeDtypeStruct((3, B, D), jnp.float32),
        ],
        compiler_params=pltpu.CompilerParams(
            dimension_semantics=("arbitrary",)),
    )(x, agg, batch3, W1, b1, bn_gamma, bn_beta, bn_mean, bn_var, W2, b2)

    res = pl.pallas_call(
        _final_kernel,
        grid=(NBLK,),
        in_specs=[
            blk, blk, bblk, _full((3, B, D)),
            _full((D,)), _full((D,)), _full((D,)), _full((D,)), _full((D,)),
        ],
        out_specs=blk,
        out_shape=jax.ShapeDtypeStruct((N, D), jnp.float32),
        compiler_params=pltpu.CompilerParams(
            dimension_semantics=("arbitrary",)),
    )(x, out, batch3, stats, ln_w, ln_b, gn_w, gn_b, gn_scale)
    return res


# speedup vs baseline: 1.3140x; 1.3140x over previous
import functools

import jax
import jax.numpy as jnp
from jax import lax
from jax.experimental import pallas as pl
from jax.experimental.pallas import tpu as pltpu
from jax.experimental.pallas import tpu_sc as plsc

N = 10000
E = 160000
D = 256
B = 64
EPS = 1e-5

NC = 2
NS = 16
LANES = 16
HALF = D // NC

EPT = E // NS
CHUNK = 80
NCHUNK = EPT // CHUNK
NPAD = 10240
ROWS = NPAD // NS
WCHUNK = 128
NWC = ROWS // WCHUNK

NB = 400
NBLK = N // NB


def _sc_body(src_hbm, dst_hbm, xlo_hbm, xhi_hbm, ea_hbm, out_hbm,
             src_v, dst_v, xrows_v, ea_v, zrow_v, acc_sh,
             sem_x, sem_e):
    c = lax.axis_index("c")
    s = lax.axis_index("s")

    zero16 = jnp.zeros((LANES,), jnp.float32)

    def zrow(r, carry):
        for j in range(HALF // LANES):
            zrow_v[r, pl.ds(j * LANES, LANES)] = zero16
        return carry

    lax.fori_loop(0, WCHUNK, zrow, 0)
    row0 = s * ROWS
    for k in range(NWC):
        pltpu.sync_copy(zrow_v, acc_sh.at[pl.ds(row0 + k * WCHUNK, WCHUNK)])
    plsc.subcore_barrier()

    e0 = s * EPT
    col0 = pl.multiple_of(c * HALF, HALF)

    def chunk_body(k, carry):
        base = e0 + k * CHUNK
        pltpu.sync_copy(src_hbm.at[pl.ds(base, CHUNK)], src_v)
        pltpu.sync_copy(dst_hbm.at[pl.ds(base, CHUNK)], dst_v)
        cp_e = pltpu.async_copy(
            ea_hbm.at[pl.ds(base, CHUNK), pl.ds(col0, HALF)], ea_v, sem_e)

        @pl.when(c == 0)
        def _():
            pltpu.async_copy(xlo_hbm.at[src_v], xrows_v, sem_x).wait()

        @pl.when(c == 1)
        def _():
            pltpu.async_copy(xhi_hbm.at[src_v], xrows_v, sem_x).wait()

        cp_e.wait()

        def rowf(r, rc):
            for j in range(HALF // LANES):
                sl = pl.ds(j * LANES, LANES)
                xrows_v[r, sl] = jnp.maximum(xrows_v[r, sl] + ea_v[r, sl], 0.0)
            return rc

        lax.fori_loop(0, CHUNK, rowf, 0)
        pltpu.sync_copy(xrows_v, acc_sh.at[dst_v], add=True)
        return carry

    lax.fori_loop(0, NCHUNK, chunk_body, 0)
    plsc.subcore_barrier()

    for k in range(NWC):
        sl = pl.ds(row0 + k * WCHUNK, WCHUNK)
        pltpu.sync_copy(acc_sh.at[sl], out_hbm.at[c, sl])


@functools.lru_cache(maxsize=None)
def _build_sc_aggregate():
    return pl.kernel(
        _sc_body,
        out_type=jax.ShapeDtypeStruct((NC, NPAD, HALF), jnp.float32),
        mesh=plsc.VectorSubcoreMesh(
            core_axis_name="c", subcore_axis_name="s",
            num_cores=NC, num_subcores=NS),
        scratch_types=[
            pltpu.VMEM((CHUNK,), jnp.int32),
            pltpu.VMEM((CHUNK,), jnp.int32),
            pltpu.VMEM((CHUNK, HALF), jnp.float32),
            pltpu.VMEM((CHUNK, HALF), jnp.float32),
            pltpu.VMEM((WCHUNK, HALF), jnp.float32),
            pltpu.VMEM_SHARED((NPAD, HALF), jnp.float32),
            pltpu.SemaphoreType.DMA,
            pltpu.SemaphoreType.DMA,
        ],
    )


def _sc_aggregate(src, dst, xlo, xhi, ea):
    return _build_sc_aggregate()(src, dst, xlo, xhi, ea)


def _mlp_stats_kernel(x_ref, agg_ref, batch_ref, w1_ref, b1_ref, g_ref,
                      be_ref, mu_ref, va_ref, w2_ref, b2_ref,
                      out_ref, stats_ref):
    i = pl.program_id(0)
    x = x_ref[...]
    h = x + jnp.concatenate([agg_ref[0], agg_ref[1]], axis=1)
    h1 = jnp.dot(h, w1_ref[...], preferred_element_type=jnp.float32)
    scale = g_ref[...] * lax.rsqrt(va_ref[...] + EPS)
    h1 = (h1 + b1_ref[...] - mu_ref[...]) * scale + be_ref[...]
    h1 = jnp.maximum(h1, 0.0)
    out = jnp.dot(h1, w2_ref[...], preferred_element_type=jnp.float32)
    out = out + b2_ref[...]
    out_ref[...] = out

    batch_col = batch_ref[0, 0, :].reshape(NB, 1)
    iota_b = lax.broadcasted_iota(jnp.int32, (NB, B), 1)
    p = (batch_col == iota_b).astype(jnp.float32)
    m1 = lax.dot_general(p, out, (((0,), (0,)), ((), ())),
                         preferred_element_type=jnp.float32)
    m2 = lax.dot_general(p, out * out, (((0,), (0,)), ((), ())),
                         preferred_element_type=jnp.float32)
    deg = jnp.broadcast_to(jnp.sum(p, axis=0)[:, None], (B, D))
    stacked = jnp.stack([m1, m2, deg])

    @pl.when(i == 0)
    def _():
        stats_ref[...] = stacked

    @pl.when(i > 0)
    def _():
        stats_ref[...] = stats_ref[...] + stacked


def _final_kernel(x_ref, out_in_ref, batch_ref, stats_ref, lnw_ref, lnb_ref,
                  gnw_ref, gnb_ref, gns_ref, res_ref):
    m1 = stats_ref[0]
    m2 = stats_ref[1]
    deg = stats_ref[2, :, 0:1]
    cnt = jnp.maximum(deg, 1.0)
    norm = cnt * D
    ms1 = jnp.sum(m1, axis=1, keepdims=True)
    ms2 = jnp.sum(m2, axis=1, keepdims=True)
    m = ms1 / norm
    varb = ms2 / norm - m * m
    inv_s = lax.rsqrt(varb + EPS)
    lnw = lnw_ref[...][None, :]
    gns = gns_ref[...][None, :]
    gnw = gnw_ref[...][None, :]
    a = lnw * inv_s
    cc = lnb_ref[...][None, :] - m * a
    mu1 = m1 / cnt
    mu2 = m2 / cnt
    beta = cc * (1.0 - gns) - a * mu1 * gns
    gvar = a * a * mu2 + 2.0 * a * beta * mu1 + beta * beta
    invt = lax.rsqrt(gvar + EPS)
    gamma = gnw * a * invt
    delta = gnw * beta * invt + gnb_ref[...][None, :]

    batch_col = batch_ref[0, 0, :].reshape(NB, 1)
    iota_b = lax.broadcasted_iota(jnp.int32, (NB, B), 1)
    p = (batch_col == iota_b).astype(jnp.float32)
    gn = jnp.dot(p, gamma, preferred_element_type=jnp.float32)
    dn = jnp.dot(p, delta, preferred_element_type=jnp.float32)
    res_ref[...] = x_ref[...] + jnp.maximum(gn * out_in_ref[...] + dn, 0.0)


def _full(shape):
    nd = len(shape)
    return pl.BlockSpec(shape, lambda i: (0,) * nd)


def kernel(x, edge_index, edge_attr, batch, W1, b1, bn_gamma, bn_beta,
           bn_mean, bn_var, W2, b2, ln_w, ln_b, gn_w, gn_b, gn_scale):
    src = edge_index[0]
    dst = edge_index[1]
    xlo = x[:, :HALF]
    xhi = x[:, HALF:]
    agg = _sc_aggregate(src, dst, xlo, xhi, edge_attr)

    batch3 = batch.reshape(NBLK, 1, NB)
    blk = pl.BlockSpec((NB, D), lambda i: (i, 0))
    bblk = pl.BlockSpec((1, 1, NB), lambda i: (i, 0, 0))

    out, stats = pl.pallas_call(
        _mlp_stats_kernel,
        grid=(NBLK,),
        in_specs=[
            blk,
            pl.BlockSpec((NC, NB, HALF), lambda i: (0, i, 0)),
            bblk,
            _full((D, 2 * D)), _full((2 * D,)), _full((2 * D,)),
            _full((2 * D,)), _full((2 * D,)), _full((2 * D,)),
            _full((2 * D, D)), _full((D,)),
        ],
        out_specs=[
            blk,
            pl.BlockSpec((3, B, D), lambda i: (0, 0, 0)),
        ],
        out_shape=[
            jax.ShapeDtypeStruct((N, D), jnp.float32),
            jax.ShapeDtypeStruct((3, B, D), jnp.float32),
        ],
        compiler_params=pltpu.CompilerParams(
            dimension_semantics=("arbitrary",)),
    )(x, agg, batch3, W1, b1, bn_gamma, bn_beta, bn_mean, bn_var, W2, b2)

    res = pl.pallas_call(
        _final_kernel,
        grid=(NBLK,),
        in_specs=[
            blk, blk, bblk, _full((3, B, D)),
            _full((D,)), _full((D,)), _full((D,)), _full((D,)), _full((D,)),
        ],
        out_specs=blk,
        out_shape=jax.ShapeDtypeStruct((N, D), jnp.float32),
        compiler_params=pltpu.CompilerParams(
            dimension_semantics=("arbitrary",)),
    )(x, out, batch3, stats, ln_w, ln_b, gn_w, gn_b, gn_scale)
    return res


# speedup vs baseline: 2.2656x; 1.7243x over previous
import functools

import jax
import jax.numpy as jnp
from jax import lax
from jax.experimental import pallas as pl
from jax.experimental.pallas import tpu as pltpu
from jax.experimental.pallas import tpu_sc as plsc

N = 10000
E = 160000
D = 256
B = 64
EPS = 1e-5

NC = 2
NS = 16
LANES = 16
HALF = D // NC

EPT = E // NS
CHUNK = 40
NCHUNK = EPT // CHUNK
NPAD = 10240
ROWS = NPAD // NS
WCHUNK = 128
NWC = ROWS // WCHUNK

NB = 400
NBLK = N // NB


def _sc_body(src_hbm, dst_hbm, xlo_hbm, xhi_hbm, ea_hbm, out_hbm,
             si0, si1, si2, si3, di0, di1, di2, di3,
             xr0, xr1, ea0, ea1, zrow_v, acc_sh,
             sx0, sx1, se0, se1, sid0, sid1, sid2, sid3):
    c = lax.axis_index("c")
    s = lax.axis_index("s")

    zero16 = jnp.zeros((LANES,), jnp.float32)

    def zrow(r, carry):
        for j in range(HALF // LANES):
            zrow_v[r, pl.ds(j * LANES, LANES)] = zero16
        return carry

    lax.fori_loop(0, WCHUNK, zrow, 0)
    row0 = s * ROWS
    for k in range(NWC):
        pltpu.sync_copy(zrow_v, acc_sh.at[pl.ds(row0 + k * WCHUNK, WCHUNK)])
    plsc.subcore_barrier()

    e0 = s * EPT
    col0 = pl.multiple_of(c * HALF, HALF)
    srcb = (si0, si1, si2, si3)
    dstb = (di0, di1, di2, di3)
    xrb = (xr0, xr1)
    eab = (ea0, ea1)
    sxb = (sx0, sx1)
    seb = (se0, se1)
    sib = (sid0, sid1, sid2, sid3)

    def idx_descs(k, j):
        sl = pl.ds(e0 + k * CHUNK, CHUNK)
        return (pltpu.make_async_copy(src_hbm.at[sl], srcb[j], sib[j]),
                pltpu.make_async_copy(dst_hbm.at[sl], dstb[j], sib[j]))

    def gth_descs(k, j):
        d = j % 2
        cp_x0 = pltpu.make_async_copy(xlo_hbm.at[srcb[j]], xrb[d], sxb[d])
        cp_x1 = pltpu.make_async_copy(xhi_hbm.at[srcb[j]], xrb[d], sxb[d])
        cp_e = pltpu.make_async_copy(
            ea_hbm.at[pl.ds(e0 + k * CHUNK, CHUNK), pl.ds(col0, HALF)],
            eab[d], seb[d])
        return cp_x0, cp_x1, cp_e

    def issue_gathers(k, j):
        cp_x0, cp_x1, cp_e = gth_descs(k, j)
        pl.when(c == 0)(cp_x0.start)
        pl.when(c == 1)(cp_x1.start)
        cp_e.start()

    def proc(k, j):
        d = j % 2
        cp_x0, cp_x1, cp_e = gth_descs(k, j)
        pl.when(c == 0)(cp_x0.wait)
        pl.when(c == 1)(cp_x1.wait)
        cp_e.wait()
        xr = xrb[d]
        ea = eab[d]

        def rowf(r, rc):
            for jj in range(HALF // LANES):
                sl = pl.ds(jj * LANES, LANES)
                xr[r, sl] = jnp.maximum(xr[r, sl] + ea[r, sl], 0.0)
            return rc

        lax.fori_loop(0, CHUNK, rowf, 0)
        pltpu.sync_copy(xr, acc_sh.at[dstb[j]], add=True)

        @pl.when(k + 4 < NCHUNK)
        def _():
            ca, cb = idx_descs(k + 4, j)
            ca.start()
            cb.start()

        @pl.when(k + 2 < NCHUNK)
        def _():
            j2 = (j + 2) % 4
            ca, cb = idx_descs(k + 2, j2)
            ca.wait()
            cb.wait()
            issue_gathers(k + 2, j2)

    for j in range(2):
        sl = pl.ds(e0 + j * CHUNK, CHUNK)
        pltpu.sync_copy(src_hbm.at[sl], srcb[j])
        pltpu.sync_copy(dst_hbm.at[sl], dstb[j])
    for j in (2, 3):
        ca, cb = idx_descs(j, j)
        ca.start()
        cb.start()
    issue_gathers(0, 0)
    issue_gathers(1, 1)

    def quad_body(i, carry):
        k = 4 * i
        proc(k, 0)
        proc(k + 1, 1)
        proc(k + 2, 2)
        proc(k + 3, 3)
        return carry

    lax.fori_loop(0, NCHUNK // 4, quad_body, 0)
    proc(NCHUNK - 2, 0)
    proc(NCHUNK - 1, 1)
    plsc.subcore_barrier()

    for k in range(NWC):
        sl = pl.ds(row0 + k * WCHUNK, WCHUNK)
        pltpu.sync_copy(acc_sh.at[sl], out_hbm.at[c, sl])


@functools.lru_cache(maxsize=None)
def _build_sc_aggregate():
    return pl.kernel(
        _sc_body,
        out_type=jax.ShapeDtypeStruct((NC, NPAD, HALF), jnp.float32),
        mesh=plsc.VectorSubcoreMesh(
            core_axis_name="c", subcore_axis_name="s",
            num_cores=NC, num_subcores=NS),
        scratch_types=[
            pltpu.VMEM((CHUNK,), jnp.int32),
            pltpu.VMEM((CHUNK,), jnp.int32),
            pltpu.VMEM((CHUNK,), jnp.int32),
            pltpu.VMEM((CHUNK,), jnp.int32),
            pltpu.VMEM((CHUNK,), jnp.int32),
            pltpu.VMEM((CHUNK,), jnp.int32),
            pltpu.VMEM((CHUNK,), jnp.int32),
            pltpu.VMEM((CHUNK,), jnp.int32),
            pltpu.VMEM((CHUNK, HALF), jnp.float32),
            pltpu.VMEM((CHUNK, HALF), jnp.float32),
            pltpu.VMEM((CHUNK, HALF), jnp.float32),
            pltpu.VMEM((CHUNK, HALF), jnp.float32),
            pltpu.VMEM((WCHUNK, HALF), jnp.float32),
            pltpu.VMEM_SHARED((NPAD, HALF), jnp.float32),
        ] + [pltpu.SemaphoreType.DMA] * 8,
    )


def _sc_aggregate(src, dst, xlo, xhi, ea):
    return _build_sc_aggregate()(src, dst, xlo, xhi, ea)


def _mlp_stats_kernel(x_ref, agg_ref, batch_ref, w1_ref, b1_ref, g_ref,
                      be_ref, mu_ref, va_ref, w2_ref, b2_ref,
                      out_ref, stats_ref):
    i = pl.program_id(0)
    x = x_ref[...]
    h = x + jnp.concatenate([agg_ref[0], agg_ref[1]], axis=1)
    h1 = jnp.dot(h, w1_ref[...], preferred_element_type=jnp.float32)
    scale = g_ref[...] * lax.rsqrt(va_ref[...] + EPS)
    h1 = (h1 + b1_ref[...] - mu_ref[...]) * scale + be_ref[...]
    h1 = jnp.maximum(h1, 0.0)
    out = jnp.dot(h1, w2_ref[...], preferred_element_type=jnp.float32)
    out = out + b2_ref[...]
    out_ref[...] = out

    batch_col = batch_ref[0, 0, :].reshape(NB, 1)
    iota_b = lax.broadcasted_iota(jnp.int32, (NB, B), 1)
    p = (batch_col == iota_b).astype(jnp.float32)
    m1 = lax.dot_general(p, out, (((0,), (0,)), ((), ())),
                         preferred_element_type=jnp.float32)
    m2 = lax.dot_general(p, out * out, (((0,), (0,)), ((), ())),
                         preferred_element_type=jnp.float32)
    deg = jnp.broadcast_to(jnp.sum(p, axis=0)[:, None], (B, D))
    stacked = jnp.stack([m1, m2, deg])

    @pl.when(i == 0)
    def _():
        stats_ref[...] = stacked

    @pl.when(i > 0)
    def _():
        stats_ref[...] = stats_ref[...] + stacked


def _final_kernel(x_ref, out_in_ref, batch_ref, stats_ref, lnw_ref, lnb_ref,
                  gnw_ref, gnb_ref, gns_ref, res_ref):
    m1 = stats_ref[0]
    m2 = stats_ref[1]
    deg = stats_ref[2, :, 0:1]
    cnt = jnp.maximum(deg, 1.0)
    norm = cnt * D
    ms1 = jnp.sum(m1, axis=1, keepdims=True)
    ms2 = jnp.sum(m2, axis=1, keepdims=True)
    m = ms1 / norm
    varb = ms2 / norm - m * m
    inv_s = lax.rsqrt(varb + EPS)
    lnw = lnw_ref[...][None, :]
    gns = gns_ref[...][None, :]
    gnw = gnw_ref[...][None, :]
    a = lnw * inv_s
    cc = lnb_ref[...][None, :] - m * a
    mu1 = m1 / cnt
    mu2 = m2 / cnt
    beta = cc * (1.0 - gns) - a * mu1 * gns
    gvar = a * a * mu2 + 2.0 * a * beta * mu1 + beta * beta
    invt = lax.rsqrt(gvar + EPS)
    gamma = gnw * a * invt
    delta = gnw * beta * invt + gnb_ref[...][None, :]

    batch_col = batch_ref[0, 0, :].reshape(NB, 1)
    iota_b = lax.broadcasted_iota(jnp.int32, (NB, B), 1)
    p = (batch_col == iota_b).astype(jnp.float32)
    gn = jnp.dot(p, gamma, preferred_element_type=jnp.float32)
    dn = jnp.dot(p, delta, preferred_element_type=jnp.float32)
    res_ref[...] = x_ref[...] + jnp.maximum(gn * out_in_ref[...] + dn, 0.0)


def _full(shape):
    nd = len(shape)
    return pl.BlockSpec(shape, lambda i: (0,) * nd)


def kernel(x, edge_index, edge_attr, batch, W1, b1, bn_gamma, bn_beta,
           bn_mean, bn_var, W2, b2, ln_w, ln_b, gn_w, gn_b, gn_scale):
    src = edge_index[0]
    dst = edge_index[1]
    xlo = x[:, :HALF]
    xhi = x[:, HALF:]
    agg = _sc_aggregate(src, dst, xlo, xhi, edge_attr)

    batch3 = batch.reshape(NBLK, 1, NB)
    blk = pl.BlockSpec((NB, D), lambda i: (i, 0))
    bblk = pl.BlockSpec((1, 1, NB), lambda i: (i, 0, 0))

    out, stats = pl.pallas_call(
        _mlp_stats_kernel,
        grid=(NBLK,),
        in_specs=[
            blk,
            pl.BlockSpec((NC, NB, HALF), lambda i: (0, i, 0)),
            bblk,
            _full((D, 2 * D)), _full((2 * D,)), _full((2 * D,)),
            _full((2 * D,)), _full((2 * D,)), _full((2 * D,)),
            _full((2 * D, D)), _full((D,)),
        ],
        out_specs=[
            blk,
            pl.BlockSpec((3, B, D), lambda i: (0, 0, 0)),
        ],
        out_shape=[
            jax.ShapeDtypeStruct((N, D), jnp.float32),
            jax.ShapeDtypeStruct((3, B, D), jnp.float32),
        ],
        compiler_params=pltpu.CompilerParams(
            dimension_semantics=("arbitrary",)),
    )(x, agg, batch3, W1, b1, bn_gamma, bn_beta, bn_mean, bn_var, W2, b2)

    res = pl.pallas_call(
        _final_kernel,
        grid=(NBLK,),
        in_specs=[
            blk, blk, bblk, _full((3, B, D)),
            _full((D,)), _full((D,)), _full((D,)), _full((D,)), _full((D,)),
        ],
        out_specs=blk,
        out_shape=jax.ShapeDtypeStruct((N, D), jnp.float32),
        compiler_params=pltpu.CompilerParams(
            dimension_semantics=("arbitrary",)),
    )(x, out, batch3, stats, ln_w, ln_b, gn_w, gn_b, gn_scale)
    return res


# speedup vs baseline: 2.3081x; 1.0187x over previous
import functools

import jax
import jax.numpy as jnp
from jax import lax
from jax.experimental import pallas as pl
from jax.experimental.pallas import tpu as pltpu
from jax.experimental.pallas import tpu_sc as plsc

N = 10000
E = 160000
D = 256
B = 64
EPS = 1e-5

NC = 2
NS = 16
LANES = 16
HALF = D // NC

EPT = E // NS
CHUNK = 40
NCHUNK = EPT // CHUNK
NPAD = 10240
ROWS = NPAD // NS
WCHUNK = 128
NWC = ROWS // WCHUNK

NB = 400
NBLK = N // NB


def _sc_body(src_hbm, dst_hbm, xlo_hbm, xhi_hbm, ea_hbm, z_hbm, out_hbm,
             si0, si1, si2, si3, di0, di1, di2, di3,
             xr0, xr1, xr2, xr3, ea0, ea1, ea2, ea3, acc_sh,
             sx0, sx1, sx2, sx3, se0, se1, se2, se3,
             ss0, ss1, ss2, ss3, sd0, sd1, sd2, sd3,
             sc0, sc1, sc2, sc3):
    c = lax.axis_index("c")
    s = lax.axis_index("s")

    row0 = s * ROWS
    for k in range(NWC):
        pltpu.sync_copy(z_hbm, acc_sh.at[pl.ds(row0 + k * WCHUNK, WCHUNK)])
    plsc.subcore_barrier()

    e0 = s * EPT
    col0 = pl.multiple_of(c * HALF, HALF)
    srcb = (si0, si1, si2, si3)
    dstb = (di0, di1, di2, di3)
    xrb = (xr0, xr1, xr2, xr3)
    eab = (ea0, ea1, ea2, ea3)
    sxb = (sx0, sx1, sx2, sx3)
    seb = (se0, se1, se2, se3)
    ssb = (ss0, ss1, ss2, ss3)
    sdb = (sd0, sd1, sd2, sd3)
    scb = (sc0, sc1, sc2, sc3)

    def src_desc(k, j):
        sl = pl.ds(e0 + k * CHUNK, CHUNK)
        return pltpu.make_async_copy(src_hbm.at[sl], srcb[j], ssb[j])

    def dst_desc(k, j):
        sl = pl.ds(e0 + k * CHUNK, CHUNK)
        return pltpu.make_async_copy(dst_hbm.at[sl], dstb[j], sdb[j])

    def gth_descs(k, j):
        cp_x0 = pltpu.make_async_copy(xlo_hbm.at[srcb[j]], xrb[j], sxb[j])
        cp_x1 = pltpu.make_async_copy(xhi_hbm.at[srcb[j]], xrb[j], sxb[j])
        cp_e = pltpu.make_async_copy(
            ea_hbm.at[pl.ds(e0 + k * CHUNK, CHUNK), pl.ds(col0, HALF)],
            eab[j], seb[j])
        return cp_x0, cp_x1, cp_e

    def issue_gathers(k, j):
        cp_x0, cp_x1, cp_e = gth_descs(k, j)
        pl.when(c == 0)(cp_x0.start)
        pl.when(c == 1)(cp_x1.start)
        cp_e.start()

    def scat_desc(j):
        return pltpu.make_async_copy(xrb[j], acc_sh.at[dstb[j]], scb[j])

    def proc(k, j):
        j2 = (j + 2) % 4
        cp_x0, cp_x1, cp_e = gth_descs(k, j)
        pl.when(c == 0)(cp_x0.wait)
        pl.when(c == 1)(cp_x1.wait)
        cp_e.wait()
        xr = xrb[j]
        ea = eab[j]

        def rowf(r, rc):
            for u in range(2):
                for jj in range(HALF // LANES):
                    sl = pl.ds(jj * LANES, LANES)
                    xr[2 * r + u, sl] = jnp.maximum(
                        xr[2 * r + u, sl] + ea[2 * r + u, sl], 0.0)
            return rc

        lax.fori_loop(0, CHUNK // 2, rowf, 0)

        @pl.when(k >= 2)
        def _():
            dst_desc(k, j).wait()
        pltpu.async_copy(xr, acc_sh.at[dstb[j]], scb[j], add=True)

        @pl.when(k + 2 < NCHUNK)
        def _():
            @pl.when(k >= 2)
            def _():
                scat_desc(j2).wait()
            dst_desc(k + 2, j2).start()
            src_desc(k + 2, j2).wait()
            issue_gathers(k + 2, j2)

        @pl.when(k + 4 < NCHUNK)
        def _():
            src_desc(k + 4, j).start()

    for j in range(2):
        sl = pl.ds(e0 + j * CHUNK, CHUNK)
        pltpu.sync_copy(src_hbm.at[sl], srcb[j])
        pltpu.sync_copy(dst_hbm.at[sl], dstb[j])
    for j in (2, 3):
        src_desc(j, j).start()
    issue_gathers(0, 0)
    issue_gathers(1, 1)

    def quad_body(i, carry):
        k = 4 * i
        proc(k, 0)
        proc(k + 1, 1)
        proc(k + 2, 2)
        proc(k + 3, 3)
        return carry

    lax.fori_loop(0, NCHUNK // 4, quad_body, 0)
    proc(NCHUNK - 2, 0)
    proc(NCHUNK - 1, 1)
    for j in range(4):
        scat_desc(j).wait()
    plsc.subcore_barrier()

    for k in range(NWC):
        sl = pl.ds(row0 + k * WCHUNK, WCHUNK)
        pltpu.sync_copy(acc_sh.at[sl], out_hbm.at[c, sl])


@functools.lru_cache(maxsize=None)
def _build_sc_aggregate():
    return pl.kernel(
        _sc_body,
        out_type=jax.ShapeDtypeStruct((NC, NPAD, HALF), jnp.float32),
        mesh=plsc.VectorSubcoreMesh(
            core_axis_name="c", subcore_axis_name="s",
            num_cores=NC, num_subcores=NS),
        scratch_types=[
            pltpu.VMEM((CHUNK,), jnp.int32),
            pltpu.VMEM((CHUNK,), jnp.int32),
            pltpu.VMEM((CHUNK,), jnp.int32),
            pltpu.VMEM((CHUNK,), jnp.int32),
            pltpu.VMEM((CHUNK,), jnp.int32),
            pltpu.VMEM((CHUNK,), jnp.int32),
            pltpu.VMEM((CHUNK,), jnp.int32),
            pltpu.VMEM((CHUNK,), jnp.int32),
            pltpu.VMEM((CHUNK, HALF), jnp.float32),
            pltpu.VMEM((CHUNK, HALF), jnp.float32),
            pltpu.VMEM((CHUNK, HALF), jnp.float32),
            pltpu.VMEM((CHUNK, HALF), jnp.float32),
            pltpu.VMEM((CHUNK, HALF), jnp.float32),
            pltpu.VMEM((CHUNK, HALF), jnp.float32),
            pltpu.VMEM((CHUNK, HALF), jnp.float32),
            pltpu.VMEM((CHUNK, HALF), jnp.float32),
            pltpu.VMEM_SHARED((NPAD, HALF), jnp.float32),
        ] + [pltpu.SemaphoreType.DMA] * 20,
    )


def _sc_aggregate(src, dst, xlo, xhi, ea):
    zeros = jnp.zeros((WCHUNK, HALF), jnp.float32)
    return _build_sc_aggregate()(src, dst, xlo, xhi, ea, zeros)


def _mlp_stats_kernel(x_ref, agg_ref, batch_ref, w1_ref, b1_ref, g_ref,
                      be_ref, mu_ref, va_ref, w2_ref, b2_ref,
                      out_ref, stats_ref):
    i = pl.program_id(0)
    x = x_ref[...]
    h = x + jnp.concatenate([agg_ref[0], agg_ref[1]], axis=1)
    h1 = jnp.dot(h, w1_ref[...], preferred_element_type=jnp.float32)
    scale = g_ref[...] * lax.rsqrt(va_ref[...] + EPS)
    h1 = (h1 + b1_ref[...] - mu_ref[...]) * scale + be_ref[...]
    h1 = jnp.maximum(h1, 0.0)
    out = jnp.dot(h1, w2_ref[...], preferred_element_type=jnp.float32)
    out = out + b2_ref[...]
    out_ref[...] = out

    batch_col = batch_ref[0, 0, :].reshape(NB, 1)
    iota_b = lax.broadcasted_iota(jnp.int32, (NB, B), 1)
    p = (batch_col == iota_b).astype(jnp.float32)
    m1 = lax.dot_general(p, out, (((0,), (0,)), ((), ())),
                         preferred_element_type=jnp.float32)
    m2 = lax.dot_general(p, out * out, (((0,), (0,)), ((), ())),
                         preferred_element_type=jnp.float32)
    deg = jnp.broadcast_to(jnp.sum(p, axis=0)[:, None], (B, D))
    stacked = jnp.stack([m1, m2, deg])

    @pl.when(i == 0)
    def _():
        stats_ref[...] = stacked

    @pl.when(i > 0)
    def _():
        stats_ref[...] = stats_ref[...] + stacked


def _final_kernel(x_ref, out_in_ref, batch_ref, stats_ref, lnw_ref, lnb_ref,
                  gnw_ref, gnb_ref, gns_ref, res_ref):
    m1 = stats_ref[0]
    m2 = stats_ref[1]
    deg = stats_ref[2, :, 0:1]
    cnt = jnp.maximum(deg, 1.0)
    norm = cnt * D
    ms1 = jnp.sum(m1, axis=1, keepdims=True)
    ms2 = jnp.sum(m2, axis=1, keepdims=True)
    m = ms1 / norm
    varb = ms2 / norm - m * m
    inv_s = lax.rsqrt(varb + EPS)
    lnw = lnw_ref[...][None, :]
    gns = gns_ref[...][None, :]
    gnw = gnw_ref[...][None, :]
    a = lnw * inv_s
    cc = lnb_ref[...][None, :] - m * a
    mu1 = m1 / cnt
    mu2 = m2 / cnt
    beta = cc * (1.0 - gns) - a * mu1 * gns
    gvar = a * a * mu2 + 2.0 * a * beta * mu1 + beta * beta
    invt = lax.rsqrt(gvar + EPS)
    gamma = gnw * a * invt
    delta = gnw * beta * invt + gnb_ref[...][None, :]

    batch_col = batch_ref[0, 0, :].reshape(NB, 1)
    iota_b = lax.broadcasted_iota(jnp.int32, (NB, B), 1)
    p = (batch_col == iota_b).astype(jnp.float32)
    gn = jnp.dot(p, gamma, preferred_element_type=jnp.float32)
    dn = jnp.dot(p, delta, preferred_element_type=jnp.float32)
    res_ref[...] = x_ref[...] + jnp.maximum(gn * out_in_ref[...] + dn, 0.0)


def _full(shape):
    nd = len(shape)
    return pl.BlockSpec(shape, lambda i: (0,) * nd)


def kernel(x, edge_index, edge_attr, batch, W1, b1, bn_gamma, bn_beta,
           bn_mean, bn_var, W2, b2, ln_w, ln_b, gn_w, gn_b, gn_scale):
    src = edge_index[0]
    dst = edge_index[1]
    xlo = x[:, :HALF]
    xhi = x[:, HALF:]
    agg = _sc_aggregate(src, dst, xlo, xhi, edge_attr)

    batch3 = batch.reshape(NBLK, 1, NB)
    blk = pl.BlockSpec((NB, D), lambda i: (i, 0))
    bblk = pl.BlockSpec((1, 1, NB), lambda i: (i, 0, 0))

    out, stats = pl.pallas_call(
        _mlp_stats_kernel,
        grid=(NBLK,),
        in_specs=[
            blk,
            pl.BlockSpec((NC, NB, HALF), lambda i: (0, i, 0)),
            bblk,
            _full((D, 2 * D)), _full((2 * D,)), _full((2 * D,)),
            _full((2 * D,)), _full((2 * D,)), _full((2 * D,)),
            _full((2 * D, D)), _full((D,)),
        ],
        out_specs=[
            blk,
            pl.BlockSpec((3, B, D), lambda i: (0, 0, 0)),
        ],
        out_shape=[
            jax.ShapeDtypeStruct((N, D), jnp.float32),
            jax.ShapeDtypeStruct((3, B, D), jnp.float32),
        ],
        compiler_params=pltpu.CompilerParams(
            dimension_semantics=("arbitrary",)),
    )(x, agg, batch3, W1, b1, bn_gamma, bn_beta, bn_mean, bn_var, W2, b2)

    res = pl.pallas_call(
        _final_kernel,
        grid=(NBLK,),
        in_specs=[
            blk, blk, bblk, _full((3, B, D)),
            _full((D,)), _full((D,)), _full((D,)), _full((D,)), _full((D,)),
        ],
        out_specs=blk,
        out_shape=jax.ShapeDtypeStruct((N, D), jnp.float32),
        compiler_params=pltpu.CompilerParams(
            dimension_semantics=("arbitrary",)),
    )(x, out, batch3, stats, ln_w, ln_b, gn_w, gn_b, gn_scale)
    return res


# speedup vs baseline: 2.4754x; 1.0725x over previous
import functools

import jax
import jax.numpy as jnp
from jax import lax
from jax.experimental import pallas as pl
from jax.experimental.pallas import tpu as pltpu
from jax.experimental.pallas import tpu_sc as plsc

N = 10000
E = 160000
D = 256
B = 64
EPS = 1e-5

NC = 2
NS = 16
LANES = 16
HALF = D // NC

EPT = E // NS
CHUNK = 40
NCHUNK = EPT // CHUNK
NPAD = 10240
ROWS = NPAD // NS
WCHUNK = 128
NWC = ROWS // WCHUNK

NB = 1000
NBLK = N // NB


def _sc_body(src_hbm, dst_hbm, xlo_hbm, xhi_hbm, ea_hbm, z_hbm, out_hbm,
             si0, si1, si2, si3, di0, di1, di2, di3,
             xr0, xr1, xr2, xr3, ea0, ea1, ea2, ea3, acc_sh,
             sx0, sx1, sx2, sx3, se0, se1, se2, se3,
             ss0, ss1, ss2, ss3, sd0, sd1, sd2, sd3,
             sc0, sc1, sc2, sc3):
    c = lax.axis_index("c")
    s = lax.axis_index("s")

    row0 = s * ROWS
    for k in range(NWC):
        pltpu.sync_copy(z_hbm, acc_sh.at[pl.ds(row0 + k * WCHUNK, WCHUNK)])
    plsc.subcore_barrier()

    e0 = s * EPT
    col0 = pl.multiple_of(c * HALF, HALF)
    srcb = (si0, si1, si2, si3)
    dstb = (di0, di1, di2, di3)
    xrb = (xr0, xr1, xr2, xr3)
    eab = (ea0, ea1, ea2, ea3)
    sxb = (sx0, sx1, sx2, sx3)
    seb = (se0, se1, se2, se3)
    ssb = (ss0, ss1, ss2, ss3)
    sdb = (sd0, sd1, sd2, sd3)
    scb = (sc0, sc1, sc2, sc3)

    def src_desc(k, j):
        sl = pl.ds(e0 + k * CHUNK, CHUNK)
        return pltpu.make_async_copy(src_hbm.at[sl], srcb[j], ssb[j])

    def dst_desc(k, j):
        sl = pl.ds(e0 + k * CHUNK, CHUNK)
        return pltpu.make_async_copy(dst_hbm.at[sl], dstb[j], sdb[j])

    def gth_descs(k, j):
        cp_x0 = pltpu.make_async_copy(xlo_hbm.at[srcb[j]], xrb[j], sxb[j])
        cp_x1 = pltpu.make_async_copy(xhi_hbm.at[srcb[j]], xrb[j], sxb[j])
        cp_e = pltpu.make_async_copy(
            ea_hbm.at[pl.ds(e0 + k * CHUNK, CHUNK), pl.ds(col0, HALF)],
            eab[j], seb[j])
        return cp_x0, cp_x1, cp_e

    def issue_gathers(k, j):
        cp_x0, cp_x1, cp_e = gth_descs(k, j)
        pl.when(c == 0)(cp_x0.start)
        pl.when(c == 1)(cp_x1.start)
        cp_e.start()

    def scat_desc(j):
        return pltpu.make_async_copy(xrb[j], acc_sh.at[dstb[j]], scb[j])

    def proc(k, j):
        j2 = (j + 2) % 4
        cp_x0, cp_x1, cp_e = gth_descs(k, j)
        pl.when(c == 0)(cp_x0.wait)
        pl.when(c == 1)(cp_x1.wait)
        cp_e.wait()
        xr = xrb[j]
        ea = eab[j]

        def rowf(r, rc):
            for u in range(2):
                for jj in range(HALF // LANES):
                    sl = pl.ds(jj * LANES, LANES)
                    xr[2 * r + u, sl] = jnp.maximum(
                        xr[2 * r + u, sl] + ea[2 * r + u, sl], 0.0)
            return rc

        lax.fori_loop(0, CHUNK // 2, rowf, 0)

        @pl.when(k >= 2)
        def _():
            dst_desc(k, j).wait()
        pltpu.async_copy(xr, acc_sh.at[dstb[j]], scb[j], add=True)

        @pl.when(k + 2 < NCHUNK)
        def _():
            @pl.when(k >= 2)
            def _():
                scat_desc(j2).wait()
            dst_desc(k + 2, j2).start()
            src_desc(k + 2, j2).wait()
            issue_gathers(k + 2, j2)

        @pl.when(k + 4 < NCHUNK)
        def _():
            src_desc(k + 4, j).start()

    for j in range(2):
        sl = pl.ds(e0 + j * CHUNK, CHUNK)
        pltpu.sync_copy(src_hbm.at[sl], srcb[j])
        pltpu.sync_copy(dst_hbm.at[sl], dstb[j])
    for j in (2, 3):
        src_desc(j, j).start()
    issue_gathers(0, 0)
    issue_gathers(1, 1)

    def quad_body(i, carry):
        k = 4 * i
        proc(k, 0)
        proc(k + 1, 1)
        proc(k + 2, 2)
        proc(k + 3, 3)
        return carry

    lax.fori_loop(0, NCHUNK // 4, quad_body, 0)
    proc(NCHUNK - 2, 0)
    proc(NCHUNK - 1, 1)
    for j in range(4):
        scat_desc(j).wait()
    plsc.subcore_barrier()

    for k in range(NWC):
        sl = pl.ds(row0 + k * WCHUNK, WCHUNK)
        pltpu.sync_copy(acc_sh.at[sl], out_hbm.at[c, sl])


@functools.lru_cache(maxsize=None)
def _build_sc_aggregate():
    return pl.kernel(
        _sc_body,
        out_type=jax.ShapeDtypeStruct((NC, NPAD, HALF), jnp.float32),
        mesh=plsc.VectorSubcoreMesh(
            core_axis_name="c", subcore_axis_name="s",
            num_cores=NC, num_subcores=NS),
        scratch_types=[
            pltpu.VMEM((CHUNK,), jnp.int32),
            pltpu.VMEM((CHUNK,), jnp.int32),
            pltpu.VMEM((CHUNK,), jnp.int32),
            pltpu.VMEM((CHUNK,), jnp.int32),
            pltpu.VMEM((CHUNK,), jnp.int32),
            pltpu.VMEM((CHUNK,), jnp.int32),
            pltpu.VMEM((CHUNK,), jnp.int32),
            pltpu.VMEM((CHUNK,), jnp.int32),
            pltpu.VMEM((CHUNK, HALF), jnp.float32),
            pltpu.VMEM((CHUNK, HALF), jnp.float32),
            pltpu.VMEM((CHUNK, HALF), jnp.float32),
            pltpu.VMEM((CHUNK, HALF), jnp.float32),
            pltpu.VMEM((CHUNK, HALF), jnp.float32),
            pltpu.VMEM((CHUNK, HALF), jnp.float32),
            pltpu.VMEM((CHUNK, HALF), jnp.float32),
            pltpu.VMEM((CHUNK, HALF), jnp.float32),
            pltpu.VMEM_SHARED((NPAD, HALF), jnp.float32),
        ] + [pltpu.SemaphoreType.DMA] * 20,
    )


def _sc_aggregate(src, dst, xlo, xhi, ea):
    zeros = jnp.zeros((WCHUNK, HALF), jnp.float32)
    return _build_sc_aggregate()(src, dst, xlo, xhi, ea, zeros)


def _mlp_stats_kernel(x_ref, agg_ref, batch_ref, w1_ref, b1_ref, g_ref,
                      be_ref, mu_ref, va_ref, w2_ref, b2_ref,
                      out_ref, stats_ref):
    i = pl.program_id(0)
    x = x_ref[...]
    h = x + jnp.concatenate([agg_ref[0], agg_ref[1]], axis=1)
    h1 = jnp.dot(h, w1_ref[...], preferred_element_type=jnp.float32)
    scale = g_ref[...] * lax.rsqrt(va_ref[...] + EPS)
    h1 = (h1 + b1_ref[...] - mu_ref[...]) * scale + be_ref[...]
    h1 = jnp.maximum(h1, 0.0)
    out = jnp.dot(h1, w2_ref[...], preferred_element_type=jnp.float32)
    out = out + b2_ref[...]
    out_ref[...] = out

    batch_col = batch_ref[0, 0, :].reshape(NB, 1)
    iota_b = lax.broadcasted_iota(jnp.int32, (NB, B), 1)
    p = (batch_col == iota_b).astype(jnp.float32)
    m1 = lax.dot_general(p, out, (((0,), (0,)), ((), ())),
                         preferred_element_type=jnp.float32)
    m2 = lax.dot_general(p, out * out, (((0,), (0,)), ((), ())),
                         preferred_element_type=jnp.float32)
    deg = jnp.broadcast_to(jnp.sum(p, axis=0)[:, None], (B, D))
    stacked = jnp.stack([m1, m2, deg])

    @pl.when(i == 0)
    def _():
        stats_ref[...] = stacked

    @pl.when(i > 0)
    def _():
        stats_ref[...] = stats_ref[...] + stacked


def _final_kernel(x_ref, out_in_ref, batch_ref, stats_ref, lnw_ref, lnb_ref,
                  gnw_ref, gnb_ref, gns_ref, res_ref):
    m1 = stats_ref[0]
    m2 = stats_ref[1]
    deg = stats_ref[2, :, 0:1]
    cnt = jnp.maximum(deg, 1.0)
    norm = cnt * D
    ms1 = jnp.sum(m1, axis=1, keepdims=True)
    ms2 = jnp.sum(m2, axis=1, keepdims=True)
    m = ms1 / norm
    varb = ms2 / norm - m * m
    inv_s = lax.rsqrt(varb + EPS)
    lnw = lnw_ref[...][None, :]
    gns = gns_ref[...][None, :]
    gnw = gnw_ref[...][None, :]
    a = lnw * inv_s
    cc = lnb_ref[...][None, :] - m * a
    mu1 = m1 / cnt
    mu2 = m2 / cnt
    beta = cc * (1.0 - gns) - a * mu1 * gns
    gvar = a * a * mu2 + 2.0 * a * beta * mu1 + beta * beta
    invt = lax.rsqrt(gvar + EPS)
    gamma = gnw * a * invt
    delta = gnw * beta * invt + gnb_ref[...][None, :]

    batch_col = batch_ref[0, 0, :].reshape(NB, 1)
    iota_b = lax.broadcasted_iota(jnp.int32, (NB, B), 1)
    p = (batch_col == iota_b).astype(jnp.float32)
    gn = jnp.dot(p, gamma, preferred_element_type=jnp.float32)
    dn = jnp.dot(p, delta, preferred_element_type=jnp.float32)
    res_ref[...] = x_ref[...] + jnp.maximum(gn * out_in_ref[...] + dn, 0.0)


def _full(shape):
    nd = len(shape)
    return pl.BlockSpec(shape, lambda i: (0,) * nd)


def kernel(x, edge_index, edge_attr, batch, W1, b1, bn_gamma, bn_beta,
           bn_mean, bn_var, W2, b2, ln_w, ln_b, gn_w, gn_b, gn_scale):
    src = edge_index[0]
    dst = edge_index[1]
    xlo = x[:, :HALF]
    xhi = x[:, HALF:]
    agg = _sc_aggregate(src, dst, xlo, xhi, edge_attr)

    batch3 = batch.reshape(NBLK, 1, NB)
    blk = pl.BlockSpec((NB, D), lambda i: (i, 0))
    bblk = pl.BlockSpec((1, 1, NB), lambda i: (i, 0, 0))

    out, stats = pl.pallas_call(
        _mlp_stats_kernel,
        grid=(NBLK,),
        in_specs=[
            blk,
            pl.BlockSpec((NC, NB, HALF), lambda i: (0, i, 0)),
            bblk,
            _full((D, 2 * D)), _full((2 * D,)), _full((2 * D,)),
            _full((2 * D,)), _full((2 * D,)), _full((2 * D,)),
            _full((2 * D, D)), _full((D,)),
        ],
        out_specs=[
            blk,
            pl.BlockSpec((3, B, D), lambda i: (0, 0, 0)),
        ],
        out_shape=[
            jax.ShapeDtypeStruct((N, D), jnp.float32),
            jax.ShapeDtypeStruct((3, B, D), jnp.float32),
        ],
        compiler_params=pltpu.CompilerParams(
            dimension_semantics=("arbitrary",)),
    )(x, agg, batch3, W1, b1, bn_gamma, bn_beta, bn_mean, bn_var, W2, b2)

    res = pl.pallas_call(
        _final_kernel,
        grid=(NBLK,),
        in_specs=[
            blk, blk, bblk, _full((3, B, D)),
            _full((D,)), _full((D,)), _full((D,)), _full((D,)), _full((D,)),
        ],
        out_specs=blk,
        out_shape=jax.ShapeDtypeStruct((N, D), jnp.float32),
        compiler_params=pltpu.CompilerParams(
            dimension_semantics=("arbitrary",)),
    )(x, out, batch3, stats, ln_w, ln_b, gn_w, gn_b, gn_scale)
    return res


# speedup vs baseline: 2.5170x; 1.0168x over previous
import functools

import jax
import jax.numpy as jnp
from jax import lax
from jax.experimental import pallas as pl
from jax.experimental.pallas import tpu as pltpu
from jax.experimental.pallas import tpu_sc as plsc

N = 10000
E = 160000
D = 256
B = 64
EPS = 1e-5

NC = 2
NS = 16
LANES = 16
HALF = D // NC

EPT = E // NS
CHUNK = 40
NCHUNK = EPT // CHUNK
NPAD = 10240
ROWS = NPAD // NS
WCHUNK = 128
NWC = ROWS // WCHUNK

NB = 2000
NBLK = N // NB


def _sc_body(src_hbm, dst_hbm, xlo_hbm, xhi_hbm, ea_hbm, z_hbm, out_hbm,
             si0, si1, si2, si3, di0, di1, di2, di3,
             xr0, xr1, xr2, xr3, ea0, ea1, ea2, ea3, acc_sh,
             sx0, sx1, sx2, sx3, se0, se1, se2, se3,
             ss0, ss1, ss2, ss3, sd0, sd1, sd2, sd3,
             sc0, sc1, sc2, sc3):
    c = lax.axis_index("c")
    s = lax.axis_index("s")

    row0 = s * ROWS
    e0 = s * EPT
    col0 = pl.multiple_of(c * HALF, HALF)
    srcb = (si0, si1, si2, si3)
    dstb = (di0, di1, di2, di3)
    xrb = (xr0, xr1, xr2, xr3)
    eab = (ea0, ea1, ea2, ea3)
    sxb = (sx0, sx1, sx2, sx3)
    seb = (se0, se1, se2, se3)
    ssb = (ss0, ss1, ss2, ss3)
    sdb = (sd0, sd1, sd2, sd3)
    scb = (sc0, sc1, sc2, sc3)

    def src_desc(k, j):
        sl = pl.ds(e0 + k * CHUNK, CHUNK)
        return pltpu.make_async_copy(src_hbm.at[sl], srcb[j], ssb[j])

    def dst_desc(k, j):
        sl = pl.ds(e0 + k * CHUNK, CHUNK)
        return pltpu.make_async_copy(dst_hbm.at[sl], dstb[j], sdb[j])

    def gth_descs(k, j):
        cp_x0 = pltpu.make_async_copy(xlo_hbm.at[srcb[j]], xrb[j], sxb[j])
        cp_x1 = pltpu.make_async_copy(xhi_hbm.at[srcb[j]], xrb[j], sxb[j])
        cp_e = pltpu.make_async_copy(
            ea_hbm.at[pl.ds(e0 + k * CHUNK, CHUNK), pl.ds(col0, HALF)],
            eab[j], seb[j])
        return cp_x0, cp_x1, cp_e

    def issue_gathers(k, j):
        cp_x0, cp_x1, cp_e = gth_descs(k, j)
        pl.when(c == 0)(cp_x0.start)
        pl.when(c == 1)(cp_x1.start)
        cp_e.start()

    def scat_desc(j):
        return pltpu.make_async_copy(xrb[j], acc_sh.at[dstb[j]], scb[j])

    def proc(k, j):
        j2 = (j + 2) % 4
        cp_x0, cp_x1, cp_e = gth_descs(k, j)
        pl.when(c == 0)(cp_x0.wait)
        pl.when(c == 1)(cp_x1.wait)
        cp_e.wait()
        xr = xrb[j]
        ea = eab[j]

        def rowf(r, rc):
            for u in range(2):
                for jj in range(HALF // LANES):
                    sl = pl.ds(jj * LANES, LANES)
                    xr[2 * r + u, sl] = jnp.maximum(
                        xr[2 * r + u, sl] + ea[2 * r + u, sl], 0.0)
            return rc

        lax.fori_loop(0, CHUNK // 2, rowf, 0)

        @pl.when(k >= 2)
        def _():
            dst_desc(k, j).wait()
        pltpu.async_copy(xr, acc_sh.at[dstb[j]], scb[j], add=True)

        @pl.when(k + 2 < NCHUNK)
        def _():
            @pl.when(k >= 2)
            def _():
                scat_desc(j2).wait()
            dst_desc(k + 2, j2).start()
            src_desc(k + 2, j2).wait()
            issue_gathers(k + 2, j2)

        @pl.when(k + 4 < NCHUNK)
        def _():
            src_desc(k + 4, j).start()

    for j in range(2):
        sl = pl.ds(e0 + j * CHUNK, CHUNK)
        pltpu.sync_copy(src_hbm.at[sl], srcb[j])
        pltpu.sync_copy(dst_hbm.at[sl], dstb[j])
    for j in (2, 3):
        src_desc(j, j).start()
    issue_gathers(0, 0)
    issue_gathers(1, 1)

    for k in range(NWC):
        pltpu.sync_copy(z_hbm, acc_sh.at[pl.ds(row0 + k * WCHUNK, WCHUNK)])
    plsc.subcore_barrier()

    def quad_body(i, carry):
        k = 4 * i
        proc(k, 0)
        proc(k + 1, 1)
        proc(k + 2, 2)
        proc(k + 3, 3)
        return carry

    lax.fori_loop(0, NCHUNK // 4, quad_body, 0)
    proc(NCHUNK - 2, 0)
    proc(NCHUNK - 1, 1)
    for j in range(4):
        scat_desc(j).wait()
    plsc.subcore_barrier()

    wdescs = []
    for k in range(NWC):
        sl = pl.ds(row0 + k * WCHUNK, WCHUNK)
        wdescs.append(
            pltpu.make_async_copy(acc_sh.at[sl], out_hbm.at[c, sl], scb[0]))
    for d in wdescs:
        d.start()
    for d in wdescs:
        d.wait()


@functools.lru_cache(maxsize=None)
def _build_sc_aggregate():
    return pl.kernel(
        _sc_body,
        out_type=jax.ShapeDtypeStruct((NC, NPAD, HALF), jnp.float32),
        mesh=plsc.VectorSubcoreMesh(
            core_axis_name="c", subcore_axis_name="s",
            num_cores=NC, num_subcores=NS),
        scratch_types=[
            pltpu.VMEM((CHUNK,), jnp.int32),
            pltpu.VMEM((CHUNK,), jnp.int32),
            pltpu.VMEM((CHUNK,), jnp.int32),
            pltpu.VMEM((CHUNK,), jnp.int32),
            pltpu.VMEM((CHUNK,), jnp.int32),
            pltpu.VMEM((CHUNK,), jnp.int32),
            pltpu.VMEM((CHUNK,), jnp.int32),
            pltpu.VMEM((CHUNK,), jnp.int32),
            pltpu.VMEM((CHUNK, HALF), jnp.float32),
            pltpu.VMEM((CHUNK, HALF), jnp.float32),
            pltpu.VMEM((CHUNK, HALF), jnp.float32),
            pltpu.VMEM((CHUNK, HALF), jnp.float32),
            pltpu.VMEM((CHUNK, HALF), jnp.float32),
            pltpu.VMEM((CHUNK, HALF), jnp.float32),
            pltpu.VMEM((CHUNK, HALF), jnp.float32),
            pltpu.VMEM((CHUNK, HALF), jnp.float32),
            pltpu.VMEM_SHARED((NPAD, HALF), jnp.float32),
        ] + [pltpu.SemaphoreType.DMA] * 20,
    )


def _sc_aggregate(src, dst, xlo, xhi, ea):
    zeros = jnp.zeros((WCHUNK, HALF), jnp.float32)
    return _build_sc_aggregate()(src, dst, xlo, xhi, ea, zeros)


def _mlp_stats_kernel(x_ref, agg_ref, batch_ref, w1_ref, b1_ref, g_ref,
                      be_ref, mu_ref, va_ref, w2_ref, b2_ref,
                      out_ref, stats_ref):
    i = pl.program_id(0)
    x = x_ref[...]
    h = x + jnp.concatenate([agg_ref[0], agg_ref[1]], axis=1)
    h1 = jnp.dot(h, w1_ref[...], preferred_element_type=jnp.float32)
    scale = g_ref[...] * lax.rsqrt(va_ref[...] + EPS)
    h1 = (h1 + b1_ref[...] - mu_ref[...]) * scale + be_ref[...]
    h1 = jnp.maximum(h1, 0.0)
    out = jnp.dot(h1, w2_ref[...], preferred_element_type=jnp.float32)
    out = out + b2_ref[...]
    out_ref[...] = out

    batch_col = batch_ref[0, 0, :].reshape(NB, 1)
    iota_b = lax.broadcasted_iota(jnp.int32, (NB, B), 1)
    p = (batch_col == iota_b).astype(jnp.float32)
    m1 = lax.dot_general(p, out, (((0,), (0,)), ((), ())),
                         preferred_element_type=jnp.float32)
    m2 = lax.dot_general(p, out * out, (((0,), (0,)), ((), ())),
                         preferred_element_type=jnp.float32)
    deg = jnp.broadcast_to(jnp.sum(p, axis=0)[:, None], (B, D))
    stacked = jnp.stack([m1, m2, deg])

    @pl.when(i == 0)
    def _():
        stats_ref[...] = stacked

    @pl.when(i > 0)
    def _():
        stats_ref[...] = stats_ref[...] + stacked


def _final_kernel(x_ref, out_in_ref, batch_ref, stats_ref, lnw_ref, lnb_ref,
                  gnw_ref, gnb_ref, gns_ref, res_ref):
    m1 = stats_ref[0]
    m2 = stats_ref[1]
    deg = stats_ref[2, :, 0:1]
    cnt = jnp.maximum(deg, 1.0)
    norm = cnt * D
    ms1 = jnp.sum(m1, axis=1, keepdims=True)
    ms2 = jnp.sum(m2, axis=1, keepdims=True)
    m = ms1 / norm
    varb = ms2 / norm - m * m
    inv_s = lax.rsqrt(varb + EPS)
    lnw = lnw_ref[...][None, :]
    gns = gns_ref[...][None, :]
    gnw = gnw_ref[...][None, :]
    a = lnw * inv_s
    cc = lnb_ref[...][None, :] - m * a
    mu1 = m1 / cnt
    mu2 = m2 / cnt
    beta = cc * (1.0 - gns) - a * mu1 * gns
    gvar = a * a * mu2 + 2.0 * a * beta * mu1 + beta * beta
    invt = lax.rsqrt(gvar + EPS)
    gamma = gnw * a * invt
    delta = gnw * beta * invt + gnb_ref[...][None, :]

    batch_col = batch_ref[0, 0, :].reshape(NB, 1)
    iota_b = lax.broadcasted_iota(jnp.int32, (NB, B), 1)
    p = (batch_col == iota_b).astype(jnp.float32)
    gn = jnp.dot(p, gamma, preferred_element_type=jnp.float32)
    dn = jnp.dot(p, delta, preferred_element_type=jnp.float32)
    res_ref[...] = x_ref[...] + jnp.maximum(gn * out_in_ref[...] + dn, 0.0)


def _full(shape):
    nd = len(shape)
    return pl.BlockSpec(shape, lambda i: (0,) * nd)


def kernel(x, edge_index, edge_attr, batch, W1, b1, bn_gamma, bn_beta,
           bn_mean, bn_var, W2, b2, ln_w, ln_b, gn_w, gn_b, gn_scale):
    src = edge_index[0]
    dst = edge_index[1]
    xlo = x[:, :HALF]
    xhi = x[:, HALF:]
    agg = _sc_aggregate(src, dst, xlo, xhi, edge_attr)

    batch3 = batch.reshape(NBLK, 1, NB)
    blk = pl.BlockSpec((NB, D), lambda i: (i, 0))
    bblk = pl.BlockSpec((1, 1, NB), lambda i: (i, 0, 0))

    out, stats = pl.pallas_call(
        _mlp_stats_kernel,
        grid=(NBLK,),
        in_specs=[
            blk,
            pl.BlockSpec((NC, NB, HALF), lambda i: (0, i, 0)),
            bblk,
            _full((D, 2 * D)), _full((2 * D,)), _full((2 * D,)),
            _full((2 * D,)), _full((2 * D,)), _full((2 * D,)),
            _full((2 * D, D)), _full((D,)),
        ],
        out_specs=[
            blk,
            pl.BlockSpec((3, B, D), lambda i: (0, 0, 0)),
        ],
        out_shape=[
            jax.ShapeDtypeStruct((N, D), jnp.float32),
            jax.ShapeDtypeStruct((3, B, D), jnp.float32),
        ],
        compiler_params=pltpu.CompilerParams(
            dimension_semantics=("arbitrary",)),
    )(x, agg, batch3, W1, b1, bn_gamma, bn_beta, bn_mean, bn_var, W2, b2)

    res = pl.pallas_call(
        _final_kernel,
        grid=(NBLK,),
        in_specs=[
            blk, blk, bblk, _full((3, B, D)),
            _full((D,)), _full((D,)), _full((D,)), _full((D,)), _full((D,)),
        ],
        out_specs=blk,
        out_shape=jax.ShapeDtypeStruct((N, D), jnp.float32),
        compiler_params=pltpu.CompilerParams(
            dimension_semantics=("arbitrary",)),
    )(x, out, batch3, stats, ln_w, ln_b, gn_w, gn_b, gn_scale)
    return res


# speedup vs baseline: 2.5853x; 1.0271x over previous
import functools

import jax
import jax.numpy as jnp
from jax import lax
from jax.experimental import pallas as pl
from jax.experimental.pallas import tpu as pltpu
from jax.experimental.pallas import tpu_sc as plsc

N = 10000
E = 160000
D = 256
B = 64
EPS = 1e-5

NC = 2
NS = 16
LANES = 16
HALF = D // NC

EPT = E // NS
CHUNK = 40
NCHUNK = EPT // CHUNK
NPAD = 10240
ROWS = NPAD // NS
WCHUNK = 128
NWC = ROWS // WCHUNK

NB = 2000
NBLK = N // NB


def _sc_body(src_hbm, dst_hbm, xlo_hbm, xhi_hbm, ea_hbm, z_hbm, out_hbm,
             si0, si1, si2, si3, di0, di1, di2, di3,
             xr0, xr1, xr2, xr3, ea0, ea1, ea2, ea3, acc_sh,
             sx0, sx1, sx2, sx3, se0, se1, se2, se3,
             ss0, ss1, ss2, ss3, sd0, sd1, sd2, sd3,
             sc0, sc1, sc2, sc3):
    c = lax.axis_index("c")
    s = lax.axis_index("s")

    row0 = s * ROWS
    e0 = s * EPT
    col0 = pl.multiple_of(c * HALF, HALF)
    srcb = (si0, si1, si2, si3)
    dstb = (di0, di1, di2, di3)
    xrb = (xr0, xr1, xr2, xr3)
    eab = (ea0, ea1, ea2, ea3)
    sxb = (sx0, sx1, sx2, sx3)
    seb = (se0, se1, se2, se3)
    ssb = (ss0, ss1, ss2, ss3)
    sdb = (sd0, sd1, sd2, sd3)
    scb = (sc0, sc1, sc2, sc3)

    def src_desc(k, j):
        sl = pl.ds(e0 + k * CHUNK, CHUNK)
        return pltpu.make_async_copy(src_hbm.at[sl], srcb[j], ssb[j])

    def dst_desc(k, j):
        sl = pl.ds(e0 + k * CHUNK, CHUNK)
        return pltpu.make_async_copy(dst_hbm.at[sl], dstb[j], sdb[j])

    def gth_descs(k, j):
        cp_x0 = pltpu.make_async_copy(xlo_hbm.at[srcb[j]], xrb[j], sxb[j])
        cp_x1 = pltpu.make_async_copy(xhi_hbm.at[srcb[j]], xrb[j], sxb[j])
        cp_e = pltpu.make_async_copy(
            ea_hbm.at[pl.ds(e0 + k * CHUNK, CHUNK), pl.ds(col0, HALF)],
            eab[j], seb[j])
        return cp_x0, cp_x1, cp_e

    def issue_gathers(k, j):
        cp_x0, cp_x1, cp_e = gth_descs(k, j)
        pl.when(c == 0)(cp_x0.start)
        pl.when(c == 1)(cp_x1.start)
        cp_e.start()

    def scat_desc(j):
        return pltpu.make_async_copy(xrb[j], acc_sh.at[dstb[j]], scb[j])

    def proc(k, j):
        j2 = (j + 2) % 4
        cp_x0, cp_x1, cp_e = gth_descs(k, j)
        pl.when(c == 0)(cp_x0.wait)
        pl.when(c == 1)(cp_x1.wait)
        cp_e.wait()
        xr = xrb[j]
        ea = eab[j]

        def rowf(r, rc):
            for u in range(2):
                for jj in range(HALF // LANES):
                    sl = pl.ds(jj * LANES, LANES)
                    xr[2 * r + u, sl] = jnp.maximum(
                        xr[2 * r + u, sl] + ea[2 * r + u, sl], 0.0)
            return rc

        lax.fori_loop(0, CHUNK // 2, rowf, 0)

        @pl.when(k >= 2)
        def _():
            dst_desc(k, j).wait()
        pltpu.async_copy(xr, acc_sh.at[dstb[j]], scb[j], add=True)

        @pl.when(k + 2 < NCHUNK)
        def _():
            @pl.when(k >= 2)
            def _():
                scat_desc(j2).wait()
            dst_desc(k + 2, j2).start()
            src_desc(k + 2, j2).wait()
            issue_gathers(k + 2, j2)

        @pl.when(k + 4 < NCHUNK)
        def _():
            src_desc(k + 4, j).start()

    for j in range(2):
        sl = pl.ds(e0 + j * CHUNK, CHUNK)
        pltpu.sync_copy(src_hbm.at[sl], srcb[j])
        pltpu.sync_copy(dst_hbm.at[sl], dstb[j])
    for j in (2, 3):
        src_desc(j, j).start()
    issue_gathers(0, 0)
    issue_gathers(1, 1)

    for k in range(NWC):
        pltpu.sync_copy(z_hbm, acc_sh.at[pl.ds(row0 + k * WCHUNK, WCHUNK)])
    plsc.subcore_barrier()

    def quad_body(i, carry):
        k = 4 * i
        proc(k, 0)
        proc(k + 1, 1)
        proc(k + 2, 2)
        proc(k + 3, 3)
        return carry

    lax.fori_loop(0, NCHUNK // 4, quad_body, 0)
    proc(NCHUNK - 2, 0)
    proc(NCHUNK - 1, 1)
    for j in range(4):
        scat_desc(j).wait()
    plsc.subcore_barrier()

    wdescs = []
    for k in range(NWC):
        sl = pl.ds(row0 + k * WCHUNK, WCHUNK)
        wdescs.append(
            pltpu.make_async_copy(acc_sh.at[sl], out_hbm.at[c, sl], scb[0]))
    for d in wdescs:
        d.start()
    for d in wdescs:
        d.wait()


@functools.lru_cache(maxsize=None)
def _build_sc_aggregate():
    return pl.kernel(
        _sc_body,
        out_type=jax.ShapeDtypeStruct((NC, NPAD, HALF), jnp.float32),
        mesh=plsc.VectorSubcoreMesh(
            core_axis_name="c", subcore_axis_name="s",
            num_cores=NC, num_subcores=NS),
        scratch_types=[
            pltpu.VMEM((CHUNK,), jnp.int32),
            pltpu.VMEM((CHUNK,), jnp.int32),
            pltpu.VMEM((CHUNK,), jnp.int32),
            pltpu.VMEM((CHUNK,), jnp.int32),
            pltpu.VMEM((CHUNK,), jnp.int32),
            pltpu.VMEM((CHUNK,), jnp.int32),
            pltpu.VMEM((CHUNK,), jnp.int32),
            pltpu.VMEM((CHUNK,), jnp.int32),
            pltpu.VMEM((CHUNK, HALF), jnp.float32),
            pltpu.VMEM((CHUNK, HALF), jnp.float32),
            pltpu.VMEM((CHUNK, HALF), jnp.float32),
            pltpu.VMEM((CHUNK, HALF), jnp.float32),
            pltpu.VMEM((CHUNK, HALF), jnp.float32),
            pltpu.VMEM((CHUNK, HALF), jnp.float32),
            pltpu.VMEM((CHUNK, HALF), jnp.float32),
            pltpu.VMEM((CHUNK, HALF), jnp.float32),
            pltpu.VMEM_SHARED((NPAD, HALF), jnp.float32),
        ] + [pltpu.SemaphoreType.DMA] * 20,
    )


def _sc_aggregate(src, dst, xlo, xhi, ea):
    zeros = jnp.zeros((WCHUNK, HALF), jnp.float32)
    return _build_sc_aggregate()(src, dst, xlo, xhi, ea, zeros)


def _fused_tc_kernel(x_ref, agg_ref, batch_ref, w1_ref, b1_ref, g_ref,
                     be_ref, mu_ref, va_ref, w2_ref, b2_ref, lnw_ref,
                     lnb_ref, gnw_ref, gnb_ref, gns_ref,
                     res_ref, out_s, stats_s):
    p = pl.program_id(0)
    i = pl.program_id(1)
    batch_col = batch_ref[0, 0, :].reshape(NB, 1)
    iota_b = lax.broadcasted_iota(jnp.int32, (NB, B), 1)
    ponehot = (batch_col == iota_b).astype(jnp.float32)
    row0 = pl.multiple_of(i * NB, NB)

    @pl.when(p == 0)
    def _():
        x = x_ref[...]
        h = x + jnp.concatenate([agg_ref[0], agg_ref[1]], axis=1)
        h1 = jnp.dot(h, w1_ref[...], preferred_element_type=jnp.float32)
        scale = g_ref[...] * lax.rsqrt(va_ref[...] + EPS)
        h1 = (h1 + b1_ref[...] - mu_ref[...]) * scale + be_ref[...]
        h1 = jnp.maximum(h1, 0.0)
        out = jnp.dot(h1, w2_ref[...], preferred_element_type=jnp.float32)
        out = out + b2_ref[...]
        out_s[pl.ds(row0, NB), :] = out
        m1 = lax.dot_general(ponehot, out, (((0,), (0,)), ((), ())),
                             preferred_element_type=jnp.float32)
        m2 = lax.dot_general(ponehot, out * out, (((0,), (0,)), ((), ())),
                             preferred_element_type=jnp.float32)
        deg = jnp.broadcast_to(jnp.sum(ponehot, axis=0)[:, None], (B, D))
        stacked = jnp.stack([m1, m2, deg])

        @pl.when(i == 0)
        def _():
            stats_s[...] = stacked

        @pl.when(i > 0)
        def _():
            stats_s[...] = stats_s[...] + stacked

    @pl.when(p == 1)
    def _():
        m1 = stats_s[0]
        m2 = stats_s[1]
        deg = stats_s[2, :, 0:1]
        cnt = jnp.maximum(deg, 1.0)
        norm = cnt * D
        ms1 = jnp.sum(m1, axis=1, keepdims=True)
        ms2 = jnp.sum(m2, axis=1, keepdims=True)
        m = ms1 / norm
        varb = ms2 / norm - m * m
        inv_s = lax.rsqrt(varb + EPS)
        lnw = lnw_ref[...][None, :]
        gns = gns_ref[...][None, :]
        gnw = gnw_ref[...][None, :]
        a = lnw * inv_s
        cc = lnb_ref[...][None, :] - m * a
        mu1 = m1 / cnt
        mu2 = m2 / cnt
        beta = cc * (1.0 - gns) - a * mu1 * gns
        gvar = a * a * mu2 + 2.0 * a * beta * mu1 + beta * beta
        invt = lax.rsqrt(gvar + EPS)
        gamma = gnw * a * invt
        delta = gnw * beta * invt + gnb_ref[...][None, :]
        gn = jnp.dot(ponehot, gamma, preferred_element_type=jnp.float32)
        dn = jnp.dot(ponehot, delta, preferred_element_type=jnp.float32)
        out = out_s[pl.ds(row0, NB), :]
        res_ref[...] = x_ref[...] + jnp.maximum(gn * out + dn, 0.0)


def _full(shape):
    nd = len(shape)
    return pl.BlockSpec(shape, lambda i: (0,) * nd)


def kernel(x, edge_index, edge_attr, batch, W1, b1, bn_gamma, bn_beta,
           bn_mean, bn_var, W2, b2, ln_w, ln_b, gn_w, gn_b, gn_scale):
    src = edge_index[0]
    dst = edge_index[1]
    xlo = x[:, :HALF]
    xhi = x[:, HALF:]
    agg = _sc_aggregate(src, dst, xlo, xhi, edge_attr)

    batch3 = batch.reshape(NBLK, 1, NB)
    blk = pl.BlockSpec((NB, D), lambda p, i: (i, 0))
    bblk = pl.BlockSpec((1, 1, NB), lambda p, i: (i, 0, 0))

    def _fullpi(shape):
        nd = len(shape)
        return pl.BlockSpec(shape, lambda p, i: (0,) * nd)

    res = pl.pallas_call(
        _fused_tc_kernel,
        grid=(2, NBLK),
        in_specs=[
            blk,
            pl.BlockSpec((NC, NB, HALF), lambda p, i: (0, (1 - p) * i, 0)),
            bblk,
            _fullpi((D, 2 * D)), _fullpi((2 * D,)), _fullpi((2 * D,)),
            _fullpi((2 * D,)), _fullpi((2 * D,)), _fullpi((2 * D,)),
            _fullpi((2 * D, D)), _fullpi((D,)),
            _fullpi((D,)), _fullpi((D,)), _fullpi((D,)), _fullpi((D,)),
            _fullpi((D,)),
        ],
        out_specs=pl.BlockSpec((NB, D), lambda p, i: (p * i, 0)),
        out_shape=jax.ShapeDtypeStruct((N, D), jnp.float32),
        scratch_shapes=[
            pltpu.VMEM((N, D), jnp.float32),
            pltpu.VMEM((3, B, D), jnp.float32),
        ],
        compiler_params=pltpu.CompilerParams(
            dimension_semantics=("arbitrary", "arbitrary")),
    )(x, agg, batch3, W1, b1, bn_gamma, bn_beta, bn_mean, bn_var, W2, b2,
      ln_w, ln_b, gn_w, gn_b, gn_scale)
    return res


# speedup vs baseline: 2.6499x; 1.0250x over previous
import functools

import jax
import jax.numpy as jnp
from jax import lax
from jax.experimental import pallas as pl
from jax.experimental.pallas import tpu as pltpu
from jax.experimental.pallas import tpu_sc as plsc

N = 10000
E = 160000
D = 256
B = 64
EPS = 1e-5

NC = 2
NS = 16
LANES = 16
HALF = D // NC

EPT = E // NS
CHUNK = 40
NCHUNK = EPT // CHUNK
NPAD = 10240
ROWS = NPAD // NS
WCHUNK = 128
NWC = ROWS // WCHUNK

NB = 2000
NBLK = N // NB


def _sc_body(src_hbm, dst_hbm, x_hbm, ea_hbm, z_hbm, out_hbm,
             si0, si1, si2, si3, di0, di1, di2, di3,
             xr0, xr1, xr2, xr3, ea0, ea1, ea2, ea3, acc_sh,
             sx0, sx1, sx2, sx3, se0, se1, se2, se3,
             ss0, ss1, ss2, ss3, sd0, sd1, sd2, sd3,
             sc0, sc1, sc2, sc3):
    c = lax.axis_index("c")
    s = lax.axis_index("s")

    row0 = s * ROWS
    e0 = s * EPT
    col0 = pl.multiple_of(c * HALF, HALF)
    srcb = (si0, si1, si2, si3)
    dstb = (di0, di1, di2, di3)
    xrb = (xr0, xr1, xr2, xr3)
    eab = (ea0, ea1, ea2, ea3)
    sxb = (sx0, sx1, sx2, sx3)
    seb = (se0, se1, se2, se3)
    ssb = (ss0, ss1, ss2, ss3)
    sdb = (sd0, sd1, sd2, sd3)
    scb = (sc0, sc1, sc2, sc3)

    def src_desc(k, j):
        sl = pl.ds(e0 + k * CHUNK, CHUNK)
        return pltpu.make_async_copy(src_hbm.at[sl], srcb[j], ssb[j])

    def dst_desc(k, j):
        sl = pl.ds(e0 + k * CHUNK, CHUNK)
        return pltpu.make_async_copy(dst_hbm.at[sl], dstb[j], sdb[j])

    def gth_descs(k, j):
        cp_x = pltpu.make_async_copy(
            x_hbm.at[srcb[j], pl.ds(col0, HALF)], xrb[j], sxb[j])
        cp_e = pltpu.make_async_copy(
            ea_hbm.at[pl.ds(e0 + k * CHUNK, CHUNK), pl.ds(col0, HALF)],
            eab[j], seb[j])
        return cp_x, cp_e

    def issue_gathers(k, j):
        cp_x, cp_e = gth_descs(k, j)
        cp_x.start()
        cp_e.start()

    def scat_desc(j):
        return pltpu.make_async_copy(xrb[j], acc_sh.at[dstb[j]], scb[j])

    def proc(k, j):
        j2 = (j + 2) % 4
        cp_x, cp_e = gth_descs(k, j)
        cp_x.wait()
        cp_e.wait()
        xr = xrb[j]
        ea = eab[j]

        def rowf(r, rc):
            for u in range(2):
                for jj in range(HALF // LANES):
                    sl = pl.ds(jj * LANES, LANES)
                    xr[2 * r + u, sl] = jnp.maximum(
                        xr[2 * r + u, sl] + ea[2 * r + u, sl], 0.0)
            return rc

        lax.fori_loop(0, CHUNK // 2, rowf, 0)

        @pl.when(k >= 2)
        def _():
            dst_desc(k, j).wait()
        pltpu.async_copy(xr, acc_sh.at[dstb[j]], scb[j], add=True)

        @pl.when(k + 2 < NCHUNK)
        def _():
            @pl.when(k >= 2)
            def _():
                scat_desc(j2).wait()
            dst_desc(k + 2, j2).start()
            src_desc(k + 2, j2).wait()
            issue_gathers(k + 2, j2)

        @pl.when(k + 4 < NCHUNK)
        def _():
            src_desc(k + 4, j).start()

    for j in range(2):
        sl = pl.ds(e0 + j * CHUNK, CHUNK)
        pltpu.sync_copy(src_hbm.at[sl], srcb[j])
        pltpu.sync_copy(dst_hbm.at[sl], dstb[j])
    for j in (2, 3):
        src_desc(j, j).start()
    issue_gathers(0, 0)
    issue_gathers(1, 1)

    for k in range(NWC):
        pltpu.sync_copy(z_hbm, acc_sh.at[pl.ds(row0 + k * WCHUNK, WCHUNK)])
    plsc.subcore_barrier()

    def quad_body(i, carry):
        k = 4 * i
        proc(k, 0)
        proc(k + 1, 1)
        proc(k + 2, 2)
        proc(k + 3, 3)
        return carry

    lax.fori_loop(0, NCHUNK // 4, quad_body, 0)
    proc(NCHUNK - 2, 0)
    proc(NCHUNK - 1, 1)
    for j in range(4):
        scat_desc(j).wait()
    plsc.subcore_barrier()

    wdescs = []
    for k in range(NWC):
        sl = pl.ds(row0 + k * WCHUNK, WCHUNK)
        wdescs.append(
            pltpu.make_async_copy(acc_sh.at[sl], out_hbm.at[c, sl], scb[0]))
    for d in wdescs:
        d.start()
    for d in wdescs:
        d.wait()


@functools.lru_cache(maxsize=None)
def _build_sc_aggregate():
    return pl.kernel(
        _sc_body,
        out_type=jax.ShapeDtypeStruct((NC, NPAD, HALF), jnp.float32),
        mesh=plsc.VectorSubcoreMesh(
            core_axis_name="c", subcore_axis_name="s",
            num_cores=NC, num_subcores=NS),
        scratch_types=[
            pltpu.VMEM((CHUNK,), jnp.int32),
            pltpu.VMEM((CHUNK,), jnp.int32),
            pltpu.VMEM((CHUNK,), jnp.int32),
            pltpu.VMEM((CHUNK,), jnp.int32),
            pltpu.VMEM((CHUNK,), jnp.int32),
            pltpu.VMEM((CHUNK,), jnp.int32),
            pltpu.VMEM((CHUNK,), jnp.int32),
            pltpu.VMEM((CHUNK,), jnp.int32),
            pltpu.VMEM((CHUNK, HALF), jnp.float32),
            pltpu.VMEM((CHUNK, HALF), jnp.float32),
            pltpu.VMEM((CHUNK, HALF), jnp.float32),
            pltpu.VMEM((CHUNK, HALF), jnp.float32),
            pltpu.VMEM((CHUNK, HALF), jnp.float32),
            pltpu.VMEM((CHUNK, HALF), jnp.float32),
            pltpu.VMEM((CHUNK, HALF), jnp.float32),
            pltpu.VMEM((CHUNK, HALF), jnp.float32),
            pltpu.VMEM_SHARED((NPAD, HALF), jnp.float32),
        ] + [pltpu.SemaphoreType.DMA] * 20,
    )


def _sc_aggregate(src, dst, x, ea):
    zeros = jnp.zeros((WCHUNK, HALF), jnp.float32)
    return _build_sc_aggregate()(src, dst, x, ea, zeros)


def _fused_tc_kernel(x_ref, agg_ref, batch_ref, w1_ref, b1_ref, g_ref,
                     be_ref, mu_ref, va_ref, w2_ref, b2_ref, lnw_ref,
                     lnb_ref, gnw_ref, gnb_ref, gns_ref,
                     res_ref, out_s, stats_s):
    p = pl.program_id(0)
    i = pl.program_id(1)
    batch_col = batch_ref[0, 0, :].reshape(NB, 1)
    iota_b = lax.broadcasted_iota(jnp.int32, (NB, B), 1)
    ponehot = (batch_col == iota_b).astype(jnp.float32)
    row0 = pl.multiple_of(i * NB, NB)

    @pl.when(p == 0)
    def _():
        x = x_ref[...]
        h = x + jnp.concatenate([agg_ref[0], agg_ref[1]], axis=1)
        h1 = jnp.dot(h, w1_ref[...], preferred_element_type=jnp.float32)
        scale = g_ref[...] * lax.rsqrt(va_ref[...] + EPS)
        h1 = (h1 + b1_ref[...] - mu_ref[...]) * scale + be_ref[...]
        h1 = jnp.maximum(h1, 0.0)
        out = jnp.dot(h1, w2_ref[...], preferred_element_type=jnp.float32)
        out = out + b2_ref[...]
        out_s[pl.ds(row0, NB), :] = out
        m1 = lax.dot_general(ponehot, out, (((0,), (0,)), ((), ())),
                             preferred_element_type=jnp.float32)
        m2 = lax.dot_general(ponehot, out * out, (((0,), (0,)), ((), ())),
                             preferred_element_type=jnp.float32)
        deg = jnp.broadcast_to(jnp.sum(ponehot, axis=0)[:, None], (B, D))
        stacked = jnp.stack([m1, m2, deg])

        @pl.when(i == 0)
        def _():
            stats_s[...] = stacked

        @pl.when(i > 0)
        def _():
            stats_s[...] = stats_s[...] + stacked

    @pl.when(p == 1)
    def _():
        m1 = stats_s[0]
        m2 = stats_s[1]
        deg = stats_s[2, :, 0:1]
        cnt = jnp.maximum(deg, 1.0)
        norm = cnt * D
        ms1 = jnp.sum(m1, axis=1, keepdims=True)
        ms2 = jnp.sum(m2, axis=1, keepdims=True)
        m = ms1 / norm
        varb = ms2 / norm - m * m
        inv_s = lax.rsqrt(varb + EPS)
        lnw = lnw_ref[...][None, :]
        gns = gns_ref[...][None, :]
        gnw = gnw_ref[...][None, :]
        a = lnw * inv_s
        cc = lnb_ref[...][None, :] - m * a
        mu1 = m1 / cnt
        mu2 = m2 / cnt
        beta = cc * (1.0 - gns) - a * mu1 * gns
        gvar = a * a * mu2 + 2.0 * a * beta * mu1 + beta * beta
        invt = lax.rsqrt(gvar + EPS)
        gamma = gnw * a * invt
        delta = gnw * beta * invt + gnb_ref[...][None, :]
        gn = jnp.dot(ponehot, gamma, preferred_element_type=jnp.float32)
        dn = jnp.dot(ponehot, delta, preferred_element_type=jnp.float32)
        out = out_s[pl.ds(row0, NB), :]
        res_ref[...] = x_ref[...] + jnp.maximum(gn * out + dn, 0.0)


def _full(shape):
    nd = len(shape)
    return pl.BlockSpec(shape, lambda i: (0,) * nd)


def kernel(x, edge_index, edge_attr, batch, W1, b1, bn_gamma, bn_beta,
           bn_mean, bn_var, W2, b2, ln_w, ln_b, gn_w, gn_b, gn_scale):
    src = edge_index[0]
    dst = edge_index[1]
    agg = _sc_aggregate(src, dst, x, edge_attr)

    batch3 = batch.reshape(NBLK, 1, NB)
    blk = pl.BlockSpec((NB, D), lambda p, i: (i, 0))
    bblk = pl.BlockSpec((1, 1, NB), lambda p, i: (i, 0, 0))

    def _fullpi(shape):
        nd = len(shape)
        return pl.BlockSpec(shape, lambda p, i: (0,) * nd)

    res = pl.pallas_call(
        _fused_tc_kernel,
        grid=(2, NBLK),
        in_specs=[
            blk,
            pl.BlockSpec((NC, NB, HALF), lambda p, i: (0, (1 - p) * i, 0)),
            bblk,
            _fullpi((D, 2 * D)), _fullpi((2 * D,)), _fullpi((2 * D,)),
            _fullpi((2 * D,)), _fullpi((2 * D,)), _fullpi((2 * D,)),
            _fullpi((2 * D, D)), _fullpi((D,)),
            _fullpi((D,)), _fullpi((D,)), _fullpi((D,)), _fullpi((D,)),
            _fullpi((D,)),
        ],
        out_specs=pl.BlockSpec((NB, D), lambda p, i: (p * i, 0)),
        out_shape=jax.ShapeDtypeStruct((N, D), jnp.float32),
        scratch_shapes=[
            pltpu.VMEM((N, D), jnp.float32),
            pltpu.VMEM((3, B, D), jnp.float32),
        ],
        compiler_params=pltpu.CompilerParams(
            dimension_semantics=("arbitrary", "arbitrary")),
    )(x, agg, batch3, W1, b1, bn_gamma, bn_beta, bn_mean, bn_var, W2, b2,
      ln_w, ln_b, gn_w, gn_b, gn_scale)
    return res
